# baseline (speedup 1.0000x reference)
import functools

import jax
import jax.numpy as jnp
from jax import lax
from jax.experimental import pallas as pl
from jax.experimental.pallas import tpu as pltpu
from jax.experimental.pallas import tpu_sc as plsc

NC, NS = 2, 16
NW = NC * NS
BATCH, HIST = 4096, 200
B = BATCH * HIST
D = 128
BPW = B // NW
CHUNK = 64
NCHUNK = BPW // CHUNK
NBUF = 8
PF = 6

assert NCHUNK % NBUF == 0 and PF < NBUF

_mesh = plsc.VectorSubcoreMesh(core_axis_name="c", subcore_axis_name="s")


@functools.partial(
    pl.kernel,
    out_type=jax.ShapeDtypeStruct((B, D), jnp.float32),
    mesh=_mesh,
    scratch_types=[
        pltpu.VMEM((NCHUNK, CHUNK), jnp.int32),
        [pltpu.VMEM((CHUNK, D), jnp.float32) for _ in range(NBUF)],
        [pltpu.SemaphoreType.DMA for _ in range(NBUF)],
        [pltpu.SemaphoreType.DMA for _ in range(NBUF)],
    ],
)
def _gather_kernel(table_hbm, idx_hbm, out_hbm, idx_v, bufs, gsems, osems):
    wid = lax.axis_index("s") * NC + lax.axis_index("c")
    base = wid * BPW
    pltpu.sync_copy(idx_hbm.at[wid], idx_v)

    def start_gather(j, b):
        pltpu.async_copy(table_hbm.at[idx_v.at[j]], bufs[b], gsems[b])

    def wait_gather(j, b):
        pltpu.make_async_copy(table_hbm.at[idx_v.at[j]], bufs[b], gsems[b]).wait()

    def start_out(j, b):
        pltpu.async_copy(bufs[b], out_hbm.at[pl.ds(base + j * CHUNK, CHUNK)],
                         osems[b])

    def wait_out(j, b):
        pltpu.make_async_copy(bufs[b], out_hbm.at[pl.ds(base + j * CHUNK, CHUNK)],
                              osems[b]).wait()

    for b in range(PF):
        start_gather(b, b)

    for b in range(NBUF):
        j = b
        wait_gather(j, b)
        start_out(j, b)
        k = j + PF
        if k < NBUF:
            start_gather(k, k)
        else:
            b2 = k % NBUF
            wait_out(k - NBUF, b2)
            start_gather(k, b2)

    @pl.loop(1, NCHUNK // NBUF - 1)
    def _group(g):
        for b in range(NBUF):
            j = g * NBUF + b
            b2 = (b + PF) % NBUF
            wait_gather(j, b)
            start_out(j, b)
            wait_out(j + PF - NBUF, b2)
            start_gather(j + PF, b2)

    for b in range(NBUF):
        j = NCHUNK - NBUF + b
        b2 = (b + PF) % NBUF
        wait_gather(j, b)
        start_out(j, b)
        k = j + PF
        if k < NCHUNK:
            wait_out(k - NBUF, b2)
            start_gather(k, b2)

    for b in range(NBUF):
        wait_out(NCHUNK - NBUF + b, b)


def kernel(token_ids, weight):
    idx = token_ids.reshape(NW, NCHUNK, CHUNK).astype(jnp.int32)
    out = _gather_kernel(weight, idx)
    return out.reshape(token_ids.shape + (D,))


# speedup vs baseline: 1.0068x; 1.0068x over previous
import functools

import jax
import jax.numpy as jnp
from jax import lax
from jax.experimental import pallas as pl
from jax.experimental.pallas import tpu as pltpu
from jax.experimental.pallas import tpu_sc as plsc

NC, NS = 2, 16
NW = NC * NS
BATCH, HIST = 4096, 200
B = BATCH * HIST
D = 128
BPW = B // NW
CHUNK = 128
NCHUNK = BPW // CHUNK
NBUF = 5
PF = 4

assert NCHUNK % NBUF == 0 and PF < NBUF

_mesh = plsc.VectorSubcoreMesh(core_axis_name="c", subcore_axis_name="s")


@functools.partial(
    pl.kernel,
    out_type=jax.ShapeDtypeStruct((B, D), jnp.float32),
    mesh=_mesh,
    scratch_types=[
        pltpu.VMEM((NCHUNK, CHUNK), jnp.int32),
        [pltpu.VMEM((CHUNK, D), jnp.float32) for _ in range(NBUF)],
        [pltpu.SemaphoreType.DMA for _ in range(NBUF)],
        [pltpu.SemaphoreType.DMA for _ in range(NBUF)],
    ],
)
def _gather_kernel(table_hbm, idx_hbm, out_hbm, idx_v, bufs, gsems, osems):
    wid = lax.axis_index("s") * NC + lax.axis_index("c")
    base = wid * BPW
    pltpu.sync_copy(idx_hbm.at[wid], idx_v)

    def start_gather(j, b):
        pltpu.async_copy(table_hbm.at[idx_v.at[j]], bufs[b], gsems[b])

    def wait_gather(j, b):
        pltpu.make_async_copy(table_hbm.at[idx_v.at[j]], bufs[b], gsems[b]).wait()

    def start_out(j, b):
        pltpu.async_copy(bufs[b], out_hbm.at[pl.ds(base + j * CHUNK, CHUNK)],
                         osems[b])

    def wait_out(j, b):
        pltpu.make_async_copy(bufs[b], out_hbm.at[pl.ds(base + j * CHUNK, CHUNK)],
                              osems[b]).wait()

    for b in range(PF):
        start_gather(b, b)

    for b in range(NBUF):
        j = b
        wait_gather(j, b)
        start_out(j, b)
        k = j + PF
        if k < NBUF:
            start_gather(k, k)
        else:
            b2 = k % NBUF
            wait_out(k - NBUF, b2)
            start_gather(k, b2)

    @pl.loop(1, NCHUNK // NBUF - 1)
    def _group(g):
        for b in range(NBUF):
            j = g * NBUF + b
            b2 = (b + PF) % NBUF
            wait_gather(j, b)
            start_out(j, b)
            wait_out(j + PF - NBUF, b2)
            start_gather(j + PF, b2)

    for b in range(NBUF):
        j = NCHUNK - NBUF + b
        b2 = (b + PF) % NBUF
        wait_gather(j, b)
        start_out(j, b)
        k = j + PF
        if k < NCHUNK:
            wait_out(k - NBUF, b2)
            start_gather(k, b2)

    for b in range(NBUF):
        wait_out(NCHUNK - NBUF + b, b)


def kernel(token_ids, weight):
    idx = token_ids.reshape(NW, NCHUNK, CHUNK).astype(jnp.int32)
    out = _gather_kernel(weight, idx)
    return out.reshape(token_ids.shape + (D,))
